# baseline (speedup 1.0000x reference)
import functools
import math

import jax
import jax.numpy as jnp
from jax import lax
from jax.experimental import pallas as pl
from jax.experimental.pallas import tpu as pltpu
from jax.experimental.pallas import tpu_sc as plsc

_NUM_CORES = 2
_NUM_SUBCORES = 16
_NUM_WORKERS = _NUM_CORES * _NUM_SUBCORES
_LANES = 16


@functools.lru_cache(maxsize=None)
def _make_gather(B, VP, D2, scale):
  D = D2 // 2
  assert B % _NUM_WORKERS == 0
  b_per_w = B // _NUM_WORKERS
  C = 320
  n_chunks = b_per_w // C
  assert b_per_w % C == 0 and C % 16 == 0 and n_chunks % 2 == 0

  mesh = plsc.VectorSubcoreMesh(core_axis_name="c", subcore_axis_name="s")

  def buf_types():
    return (
        pltpu.VMEM((C,), jnp.int32),
        pltpu.VMEM((C,), jnp.int32),
        pltpu.VMEM((C,), jnp.int32),
        pltpu.VMEM((C, D2), jnp.float32),
        pltpu.VMEM((C // 2, D2), jnp.float32),
        pltpu.SemaphoreType.DMA,
        pltpu.SemaphoreType.DMA,
    )

  @functools.partial(
      pl.kernel,
      mesh=mesh,
      out_type=jax.ShapeDtypeStruct((B // 2, D2), jnp.float32),
      scratch_types=[buf_types(), buf_types()],
      compiler_params=pltpu.CompilerParams(use_tc_tiling_on_sc=False),
  )
  def gather_kernel(table_hbm, idx_hbm, out_hbm, buf0, buf1):
    wid = lax.axis_index("s") * _NUM_CORES + lax.axis_index("c")
    base = wid * b_per_w
    bufs = (buf0, buf1)

    def fetch(g, b):
      tok_v, gidx_v, poff_v, rows_v, _, gsem, _ = bufs[b]
      off = base + g * C
      pltpu.sync_copy(idx_hbm.at[pl.ds(off, C)], tok_v)

      @plsc.parallel_loop(0, C // _LANES, step=1, unroll=4)
      def _(k):
        sl = pl.ds(k * _LANES, _LANES)
        t = tok_v[sl]
        gidx_v[sl] = t >> 1
        poff_v[sl] = (t & 1) * D

      pltpu.async_copy(table_hbm.at[gidx_v], rows_v, gsem)

    def select_store(g, b):
      _, gidx_v, poff_v, rows_v, out_v, gsem, ssem = bufs[b]
      pltpu.make_async_copy(table_hbm.at[gidx_v], rows_v, gsem).wait()

      @plsc.parallel_loop(0, C // _LANES, step=1, unroll=2)
      def _(k):
        pvec = poff_v[pl.ds(k * _LANES, _LANES)]
        for m in range(_LANES // 2):
          p0 = pvec[2 * m]
          p1 = pvec[2 * m + 1]
          q = k * (_LANES // 2) + m
          r = k * _LANES + 2 * m
          for j in range(D // _LANES):
            jo = j * _LANES
            out_v[q, pl.ds(jo, _LANES)] = (
                rows_v[r, pl.ds(p0 + jo, _LANES)] * scale)
            out_v[q, pl.ds(D + jo, _LANES)] = (
                rows_v[r + 1, pl.ds(p1 + jo, _LANES)] * scale)

      row_off = (base + g * C) // 2
      pltpu.async_copy(out_v, out_hbm.at[pl.ds(row_off, C // 2)], ssem)

    def wait_store(g, b):
      _, _, _, _, out_v, _, ssem = bufs[b]
      row_off = (base + g * C) // 2
      pltpu.make_async_copy(out_v, out_hbm.at[pl.ds(row_off, C // 2)],
                            ssem).wait()

    fetch(0, 0)

    def do_pair(p, carry):
      g0 = p * 2

      @pl.when(p > 0)
      def _():
        wait_store(g0 - 1, 1)

      fetch(g0 + 1, 1)
      select_store(g0, 0)

      @pl.when(g0 + 2 < n_chunks)
      def _():
        wait_store(g0, 0)
        fetch(g0 + 2, 0)

      select_store(g0 + 1, 1)
      return carry

    lax.fori_loop(0, n_chunks // 2, do_pair, 0)

    wait_store(n_chunks - 2, 0)
    wait_store(n_chunks - 1, 1)

  return gather_kernel


def kernel(tokens, table):
  bsz, hist = tokens.shape
  vocab, emb = table.shape
  scale = float(math.sqrt(emb))
  B = bsz * hist
  flat = tokens.reshape(B).astype(jnp.int32)
  paired = table.reshape(vocab // 2, 2 * emb)
  out = _make_gather(B, vocab // 2, 2 * emb, scale)(paired, flat)
  return out.reshape(bsz, hist, emb)


# speedup vs baseline: 1.2291x; 1.2291x over previous
import functools
import math

import jax
import jax.numpy as jnp
from jax import lax
from jax.experimental import pallas as pl
from jax.experimental.pallas import tpu as pltpu
from jax.experimental.pallas import tpu_sc as plsc

_NUM_CORES = 2
_NUM_SUBCORES = 16
_NUM_WORKERS = _NUM_CORES * _NUM_SUBCORES
_LANES = 16


@functools.lru_cache(maxsize=None)
def _make_gather(B, VP, D2, scale):
  D = D2 // 2
  assert B % _NUM_WORKERS == 0
  b_per_w = B // _NUM_WORKERS
  C = 160
  n_chunks = b_per_w // C
  assert b_per_w % C == 0 and C % _LANES == 0 and n_chunks % 2 == 0

  mesh = plsc.VectorSubcoreMesh(core_axis_name="c", subcore_axis_name="s")

  def buf_types():
    return (
        pltpu.VMEM((C,), jnp.int32),
        pltpu.VMEM((C,), jnp.int32),
        pltpu.VMEM((C, D2), jnp.float32),
        pltpu.VMEM((C, D), jnp.float32),
        pltpu.SemaphoreType.DMA,
        pltpu.SemaphoreType.DMA,
    )

  @functools.partial(
      pl.kernel,
      mesh=mesh,
      out_type=jax.ShapeDtypeStruct((B, D), jnp.float32),
      scratch_types=[
          pltpu.VMEM((b_per_w,), jnp.int32),
          buf_types(),
          buf_types(),
      ],
  )
  def gather_kernel(table_hbm, idx_hbm, out_hbm, tok_full, buf0, buf1):
    wid = lax.axis_index("s") * _NUM_CORES + lax.axis_index("c")
    base = pl.multiple_of(wid * b_per_w, 1024)
    bufs = (buf0, buf1)

    pltpu.sync_copy(idx_hbm.at[pl.ds(base, b_per_w)], tok_full)

    def fetch(g, b):
      gidx_v, poff_v, rows_v, _, gsem, _ = bufs[b]

      @plsc.parallel_loop(0, C // _LANES, step=1, unroll=4)
      def _(k):
        sl = pl.ds(k * _LANES, _LANES)
        t = tok_full[pl.ds(g * C + k * _LANES, _LANES)]
        gidx_v[sl] = t >> 1
        poff_v[sl] = (t & 1) * D

      pltpu.async_copy(table_hbm.at[gidx_v], rows_v, gsem)

    def select_store(g, b):
      gidx_v, poff_v, rows_v, out_v, gsem, ssem = bufs[b]
      pltpu.make_async_copy(table_hbm.at[gidx_v], rows_v, gsem).wait()

      @plsc.parallel_loop(0, C // _LANES, step=1, unroll=2)
      def _(k):
        pvec = poff_v[pl.ds(k * _LANES, _LANES)]
        for m in range(_LANES):
          p = pvec[m]
          t = k * _LANES + m
          for j in range(D // _LANES):
            jo = j * _LANES
            out_v[t, pl.ds(jo, _LANES)] = (
                rows_v[t, pl.ds(p + jo, _LANES)] * scale)

      off = pl.multiple_of(base + g * C, 32)
      pltpu.async_copy(out_v, out_hbm.at[pl.ds(off, C)], ssem)

    def wait_store(g, b):
      _, _, _, out_v, _, ssem = bufs[b]
      off = pl.multiple_of(base + g * C, 32)
      pltpu.make_async_copy(out_v, out_hbm.at[pl.ds(off, C)], ssem).wait()

    fetch(0, 0)

    def do_pair(p, carry):
      g0 = p * 2

      @pl.when(p > 0)
      def _():
        wait_store(g0 - 1, 1)

      fetch(g0 + 1, 1)
      select_store(g0, 0)

      @pl.when(g0 + 2 < n_chunks)
      def _():
        wait_store(g0, 0)
        fetch(g0 + 2, 0)

      select_store(g0 + 1, 1)
      return carry

    lax.fori_loop(0, n_chunks // 2, do_pair, 0)

    wait_store(n_chunks - 2, 0)
    wait_store(n_chunks - 1, 1)

  return gather_kernel


def kernel(tokens, table):
  bsz, hist = tokens.shape
  vocab, emb = table.shape
  scale = float(math.sqrt(emb))
  B = bsz * hist
  flat = tokens.reshape(B).astype(jnp.int32)
  paired = table.reshape(vocab // 2, 2 * emb)
  out = _make_gather(B, vocab // 2, 2 * emb, scale)(paired, flat)
  return out.reshape(bsz, hist, emb)
